# baseline (speedup 1.0000x reference)
import functools

import jax
import jax.numpy as jnp
import numpy as np
from jax import lax
from jax.experimental import pallas as pl
from jax.experimental.pallas import tpu as pltpu
from jax.experimental.pallas import tpu_sc as plsc

N_SCENE = 10000
K = 4096
GEO = 256
GIN = 192
HID = 256
ED = 16
H = 8
D = 32
HD = H * D
L = 2
E_OBS = 131072
E_CRD = 16384

NW = 32
SC_CHUNK = 128

_f32 = jnp.float32
_i32 = jnp.int32


def _proj_body(nout, packed, x_ref, w_ref, b_ref, *out_refs):
    acc = jnp.dot(x_ref[...], w_ref[...], preferred_element_type=_f32)
    acc = acc + b_ref[...]
    for j in range(nout):
        sl = acc[:, j * HID:(j + 1) * HID]
        if packed:
            h0 = sl[:, :HID // 2].astype(jnp.bfloat16)[:, None, :]
            h1 = sl[:, HID // 2:].astype(jnp.bfloat16)[:, None, :]
            w = pltpu.bitcast(jnp.concatenate([h0, h1], axis=1), _i32)
            out_refs[j][...] = w[:, 0, :]
        else:
            out_refs[j][...] = sl


def _proj(x, w, b, nout, bn, packed=False):
    n, kin = x.shape
    grid = (n // bn,)
    ow = HID // 2 if packed else HID
    odt = _i32 if packed else _f32
    out = pl.pallas_call(
        functools.partial(_proj_body, nout, packed),
        grid=grid,
        in_specs=[
            pl.BlockSpec((bn, kin), lambda i: (i, 0)),
            pl.BlockSpec((kin, nout * HID), lambda i: (0, 0)),
            pl.BlockSpec((1, nout * HID), lambda i: (0, 0)),
        ],
        out_specs=[pl.BlockSpec((bn, ow), lambda i: (i, 0))
                   for _ in range(nout)],
        out_shape=[jax.ShapeDtypeStruct((n, ow), odt) for _ in range(nout)],
    )(x, w, b)
    return out


def _unpack(ref):
    bf = pltpu.bitcast(ref[...][:, None, :], jnp.bfloat16)
    return bf[:, 0, :].astype(_f32), bf[:, 1, :].astype(_f32)


def _edge_body(qg_ref, kg_ref, vg_ref, ea_ref, we_ref, sel_ref, spread_ref,
               pvx_ref):
    hw = HD // 2
    eproj = jnp.dot(ea_ref[...], we_ref[...], preferred_element_type=_f32)
    e0, e1 = eproj[:, :hw], eproj[:, hw:]
    q0, q1 = _unpack(qg_ref)
    k0, k1 = _unpack(kg_ref)
    prod0 = q0 * (k0 + e0)
    prod1 = q1 * (k1 + e1)
    sel = sel_ref[...]
    logits = (jnp.dot(prod0, sel[:hw], preferred_element_type=_f32)
              + jnp.dot(prod1, sel[hw:], preferred_element_type=_f32))
    p = jnp.exp(logits * (1.0 / np.sqrt(D)))
    pexp = jnp.dot(p, spread_ref[...], preferred_element_type=_f32)
    v0, v1 = _unpack(vg_ref)
    pv0 = (v0 + e0) * pexp[:, :hw]
    pv1 = (v1 + e1) * pexp[:, hw:]
    be = p.shape[0]
    pvx_ref[...] = jnp.concatenate(
        [pv0, pv1, p, jnp.zeros((be, 128 - 16), _f32)], axis=-1)


def _edge(qg, kg, vg, ea, we, sel, spread, be):
    e = qg.shape[0]
    grid = (e // be,)
    return pl.pallas_call(
        _edge_body,
        grid=grid,
        in_specs=[
            pl.BlockSpec((be, HD // 2), lambda i: (i, 0)),
            pl.BlockSpec((be, HD // 2), lambda i: (i, 0)),
            pl.BlockSpec((be, HD // 2), lambda i: (i, 0)),
            pl.BlockSpec((be, ED), lambda i: (i, 0)),
            pl.BlockSpec((ED, HD), lambda i: (0, 0)),
            pl.BlockSpec((HD, 16), lambda i: (0, 0)),
            pl.BlockSpec((16, HD), lambda i: (0, 0)),
        ],
        out_specs=pl.BlockSpec((be, HD + 128), lambda i: (i, 0)),
        out_shape=jax.ShapeDtypeStruct((e, HD + 128), _f32),
    )(qg, kg, vg, ea, we, sel, spread)


def _combine_body(h_ref, ao_ref, so_ref, ac_ref, sc_ref, wo_ref,
                  spread_ref, lns_ref, lnb_ref, out_ref):
    spread = spread_ref[...]
    so = jnp.dot((so_ref[0] + so_ref[1])[:, :16], spread,
                 preferred_element_type=_f32)
    sc = jnp.dot((sc_ref[0] + sc_ref[1])[:, :16], spread,
                 preferred_element_type=_f32)
    m = (ao_ref[0] + ao_ref[1]) / (so + 1e-9) \
        + (ac_ref[0] + ac_ref[1]) / (sc + 1e-9)
    z = h_ref[...] + jnp.dot(m, wo_ref[...], preferred_element_type=_f32)
    mu = jnp.mean(z, axis=-1, keepdims=True)
    zc = z - mu
    var = jnp.mean(zc * zc, axis=-1, keepdims=True)
    out_ref[...] = zc * lax.rsqrt(var + 1e-5) * lns_ref[...] + lnb_ref[...]


def _combine(h, agg_o, s_o, agg_c, s_c, wo, spread, lns, lnb, bn):
    n = h.shape[0]
    grid = (n // bn,)
    return pl.pallas_call(
        _combine_body,
        grid=grid,
        in_specs=[
            pl.BlockSpec((bn, HID), lambda i: (i, 0)),
            pl.BlockSpec((2, bn, HD), lambda i: (0, i, 0)),
            pl.BlockSpec((2, bn, 128), lambda i: (0, i, 0)),
            pl.BlockSpec((2, bn, HD), lambda i: (0, i, 0)),
            pl.BlockSpec((2, bn, 128), lambda i: (0, i, 0)),
            pl.BlockSpec((HD, HID), lambda i: (0, 0)),
            pl.BlockSpec((16, HD), lambda i: (0, 0)),
            pl.BlockSpec((1, HID), lambda i: (0, 0)),
            pl.BlockSpec((1, HID), lambda i: (0, 0)),
        ],
        out_specs=pl.BlockSpec((bn, HID), lambda i: (i, 0)),
        out_shape=jax.ShapeDtypeStruct((n, HID), _f32),
    )(h, agg_o, s_o, agg_c, s_c, wo, spread, lns, lnb)


@functools.lru_cache(maxsize=None)
def _make_gather(e):
    per_w = e // NW
    c = 64
    n2 = per_w // (2 * c)
    hw = HD // 2
    mesh = plsc.VectorSubcoreMesh(core_axis_name="c", subcore_axis_name="s")

    @functools.partial(
        pl.kernel,
        out_type=[jax.ShapeDtypeStruct((e, hw), _i32)] * 3,
        mesh=mesh,
        scratch_types=[
            pltpu.VMEM((c,), _i32), pltpu.VMEM((c,), _i32),
            pltpu.VMEM((c,), _i32), pltpu.VMEM((c,), _i32),
            pltpu.VMEM((c, hw), _i32), pltpu.VMEM((c, hw), _i32),
            pltpu.VMEM((c, hw), _i32),
            pltpu.VMEM((c, hw), _i32), pltpu.VMEM((c, hw), _i32),
            pltpu.VMEM((c, hw), _i32),
            pltpu.SemaphoreType.DMA, pltpu.SemaphoreType.DMA,
            pltpu.SemaphoreType.DMA, pltpu.SemaphoreType.DMA,
            pltpu.SemaphoreType.DMA, pltpu.SemaphoreType.DMA,
        ],
    )
    def gather(qtab, ktab, vtab, dst, src, qg, kg, vg,
               d0, s0, d1, s1, q0, k0, v0, q1, k1, v1,
               sq0, sk0, sv0, sq1, sk1, sv1):
        wid = lax.axis_index("c") * 16 + lax.axis_index("s")
        base = wid * per_w
        dbuf = (d0, d1)
        sbuf = (s0, s1)
        rows = ((q0, k0, v0), (q1, k1, v1))
        sems = ((sq0, sk0, sv0), (sq1, sk1, sv1))

        def load_idx(off, b):
            pltpu.sync_copy(dst.at[pl.ds(off, c)], dbuf[b])
            pltpu.sync_copy(src.at[pl.ds(off, c)], sbuf[b])

        def issue(b):
            pltpu.async_copy(qtab.at[dbuf[b]], rows[b][0], sems[b][0])
            pltpu.async_copy(ktab.at[sbuf[b]], rows[b][1], sems[b][1])
            pltpu.async_copy(vtab.at[sbuf[b]], rows[b][2], sems[b][2])

        def wait_and_write(off, b):
            pltpu.make_async_copy(qtab.at[dbuf[b]], rows[b][0], sems[b][0]).wait()
            pltpu.make_async_copy(ktab.at[sbuf[b]], rows[b][1], sems[b][1]).wait()
            pltpu.make_async_copy(vtab.at[sbuf[b]], rows[b][2], sems[b][2]).wait()
            pltpu.sync_copy(rows[b][0], qg.at[pl.ds(off, c)])
            pltpu.sync_copy(rows[b][1], kg.at[pl.ds(off, c)])
            pltpu.sync_copy(rows[b][2], vg.at[pl.ds(off, c)])

        load_idx(base, 0)
        issue(0)

        def body(j, carry):
            off0 = base + (2 * j) * c
            load_idx(off0 + c, 1)
            issue(1)
            wait_and_write(off0, 0)

            @pl.when(j + 1 < n2)
            def _():
                load_idx(off0 + 2 * c, 0)
                issue(0)

            wait_and_write(off0 + c, 1)
            return carry

        lax.fori_loop(0, n2, body, 0)

    return gather


@functools.lru_cache(maxsize=None)
def _make_scatter(e):
    per_w = e // NW
    c = 64
    n_chunks = per_w // c
    rows_per_tile = K // 16
    hw = 128
    mesh = plsc.VectorSubcoreMesh(core_axis_name="c", subcore_axis_name="s")

    @functools.partial(
        pl.kernel,
        out_type=[
            jax.ShapeDtypeStruct((2, K, HD), _f32),
            jax.ShapeDtypeStruct((2, K, hw), _f32),
        ],
        mesh=mesh,
        scratch_types=[
            pltpu.VMEM((c,), _i32), pltpu.VMEM((c,), _i32),
            pltpu.VMEM((c, hw), _f32), pltpu.VMEM((c, hw), _f32),
            pltpu.VMEM((c, hw), _f32), pltpu.VMEM((c, hw), _f32),
            pltpu.VMEM_SHARED((K, hw), _f32),
            pltpu.VMEM_SHARED((K, hw), _f32),
            pltpu.SemaphoreType.DMA, pltpu.SemaphoreType.DMA,
            pltpu.SemaphoreType.DMA, pltpu.SemaphoreType.DMA,
            pltpu.SemaphoreType.DMA, pltpu.SemaphoreType.DMA,
        ],
    )
    def scatter(pvx, dst, zero_big, agg2, s2,
                d0, d1, a0, b0, a1, b1, acc0, acc1,
                sd0, sa0, sb0, sd1, sa1, sb1):
        cid = lax.axis_index("c")
        sid = lax.axis_index("s")
        wid = cid * 16 + sid
        row0 = sid * rows_per_tile
        base = wid * per_w
        dbuf = (d0, d1)
        bufa = (a0, a1)
        bufb = (b0, b1)
        sems = ((sd0, sa0, sb0), (sd1, sa1, sb1))
        n2 = n_chunks // 2

        def run_phase(groups, accs):
            ng = len(groups)

            def load(off, s):
                pltpu.async_copy(dst.at[pl.ds(off, c)], dbuf[s], sems[s][0])
                pltpu.async_copy(pvx.at[pl.ds(off, c),
                                        pl.ds(groups[0] * hw, hw)],
                                 bufa[s], sems[s][1])
                if ng > 1:
                    pltpu.async_copy(pvx.at[pl.ds(off, c),
                                            pl.ds(groups[1] * hw, hw)],
                                     bufb[s], sems[s][2])

            def wait_and_add(off, s):
                pltpu.make_async_copy(dst.at[pl.ds(off, c)], dbuf[s],
                                      sems[s][0]).wait()
                pltpu.make_async_copy(pvx.at[pl.ds(off, c),
                                             pl.ds(groups[0] * hw, hw)],
                                      bufa[s], sems[s][1]).wait()
                if ng > 1:
                    pltpu.make_async_copy(pvx.at[pl.ds(off, c),
                                                 pl.ds(groups[1] * hw, hw)],
                                          bufb[s], sems[s][2]).wait()
                pltpu.sync_copy(bufa[s], accs[0].at[dbuf[s]], add=True)
                if ng > 1:
                    pltpu.sync_copy(bufb[s], accs[1].at[dbuf[s]], add=True)

            load(base, 0)

            def body(j, carry):
                off0 = base + (2 * j) * c
                load(off0 + c, 1)
                wait_and_add(off0, 0)

                @pl.when(j + 1 < n2)
                def _():
                    load(off0 + 2 * c, 0)

                wait_and_add(off0 + c, 1)
                return carry

            lax.fori_loop(0, n2, body, 0)

        pltpu.sync_copy(zero_big, acc0.at[pl.ds(row0, rows_per_tile)])
        pltpu.sync_copy(zero_big, acc1.at[pl.ds(row0, rows_per_tile)])
        plsc.subcore_barrier()
        run_phase((0, 1), (acc0, acc1))
        plsc.subcore_barrier()
        pltpu.sync_copy(acc0.at[pl.ds(row0, rows_per_tile)],
                        agg2.at[cid, pl.ds(row0, rows_per_tile), pl.ds(0, hw)])
        pltpu.sync_copy(acc1.at[pl.ds(row0, rows_per_tile)],
                        agg2.at[cid, pl.ds(row0, rows_per_tile), pl.ds(hw, hw)])
        plsc.subcore_barrier()

        pltpu.sync_copy(zero_big, acc0.at[pl.ds(row0, rows_per_tile)])
        plsc.subcore_barrier()
        run_phase((2,), (acc0,))
        plsc.subcore_barrier()
        pltpu.sync_copy(acc0.at[pl.ds(row0, rows_per_tile)],
                        s2.at[cid, pl.ds(row0, rows_per_tile)])

    return scatter


def _conv(q, ktab, vtab, src, dst, ea, we, sel, spread, zb):
    e = src.shape[0]
    qg, kg, vg = _make_gather(e)(q, ktab, vtab, dst, src)
    pvx = _edge(qg, kg, vg, ea, we, sel, spread, be=1024)
    agg2, s2 = _make_scatter(e)(pvx, dst, zb)
    return agg2, s2


def kernel(scene_x, gripper_left_x, gripper_right_x, edge_index_obs_left,
           edge_index_obs_right, edge_index_coord_lr, edge_index_coord_rl,
           edge_attr_obs_left, edge_attr_obs_right, edge_attr_coord_lr,
           edge_attr_coord_rl, scene_W, scene_b, grip_W, grip_b,
           Wq, Wk, Wv, We, Wo, ln_s, ln_b):
    sel = jnp.zeros((HD, 16), _f32).at[jnp.arange(HD), jnp.arange(HD) // D].set(1.0)
    spread = sel.T[:, :]
    zb = jnp.zeros((K // 16, 128), _f32)

    src_ol, dst_ol = edge_index_obs_left[0], edge_index_obs_left[1]
    src_or, dst_or = edge_index_obs_right[0], edge_index_obs_right[1]
    src_lr, dst_lr = edge_index_coord_lr[0], edge_index_coord_lr[1]
    src_rl, dst_rl = edge_index_coord_rl[0], edge_index_coord_rl[1]

    (scene,) = _proj(scene_x, scene_W, scene_b[None, :], 1, bn=400)
    (left,) = _proj(gripper_left_x, grip_W, grip_b[None, :], 1, bn=512)
    (right,) = _proj(gripper_right_x, grip_W, grip_b[None, :], 1, bn=512)

    zbias1 = jnp.zeros((1, 4 * HID), _f32)
    for l in range(L):
        k0, v0, k1, v1 = _proj(
            scene, jnp.concatenate([Wk[l, 0], Wv[l, 0], Wk[l, 1], Wv[l, 1]], axis=1),
            zbias1, 4, bn=400, packed=True)
        q0, q3, k2, v2 = _proj(
            left, jnp.concatenate([Wq[l, 0], Wq[l, 3], Wk[l, 2], Wv[l, 2]], axis=1),
            zbias1, 4, bn=512, packed=True)
        q1, q2, k3, v3 = _proj(
            right, jnp.concatenate([Wq[l, 1], Wq[l, 2], Wk[l, 3], Wv[l, 3]], axis=1),
            zbias1, 4, bn=512, packed=True)

        ao_l, so_l = _conv(q0, k0, v0, src_ol, dst_ol, edge_attr_obs_left,
                           We[l, 0], sel, spread, zb)
        ac_l, sc_l = _conv(q3, k3, v3, src_rl, dst_rl, edge_attr_coord_rl,
                           We[l, 3], sel, spread, zb)
        ao_r, so_r = _conv(q1, k1, v1, src_or, dst_or, edge_attr_obs_right,
                           We[l, 1], sel, spread, zb)
        ac_r, sc_r = _conv(q2, k2, v2, src_lr, dst_lr, edge_attr_coord_lr,
                           We[l, 2], sel, spread, zb)

        new_left = _combine(left, ao_l, so_l, ac_l, sc_l, Wo[l, 1], spread,
                            ln_s[l, 1][None, :], ln_b[l, 1][None, :], bn=512)
        new_right = _combine(right, ao_r, so_r, ac_r, sc_r, Wo[l, 2], spread,
                             ln_s[l, 2][None, :], ln_b[l, 2][None, :], bn=512)
        left, right = new_left, new_right

    return (left, right)


# speedup vs baseline: 1.5251x; 1.5251x over previous
import functools

import jax
import jax.numpy as jnp
import numpy as np
from jax import lax
from jax.experimental import pallas as pl
from jax.experimental.pallas import tpu as pltpu
from jax.experimental.pallas import tpu_sc as plsc

N_SCENE = 10000
K = 4096
GEO = 256
GIN = 192
HID = 256
ED = 16
H = 8
D = 32
HD = H * D
L = 2
E_OBS = 131072
E_CRD = 16384

NW = 32
SC_CHUNK = 128

_f32 = jnp.float32
_i32 = jnp.int32


def _proj_body(nout, packed, x_ref, w_ref, b_ref, *out_refs):
    acc = jnp.dot(x_ref[...], w_ref[...], preferred_element_type=_f32)
    acc = acc + b_ref[...]
    for j in range(nout):
        sl = acc[:, j * HID:(j + 1) * HID]
        if packed:
            a = jax.lax.bitcast_convert_type(sl[:, :HID // 2], _i32)
            b = jax.lax.bitcast_convert_type(sl[:, HID // 2:], _i32)
            lo = jax.lax.shift_right_logical(a + 0x8000, 16)
            hi = (b + 0x8000) & jnp.int32(-65536)
            out_refs[j][...] = lo | hi
        else:
            out_refs[j][...] = sl


def _proj(x, w, b, nout, bn, packed=False):
    n, kin = x.shape
    grid = (n // bn,)
    ow = HID // 2 if packed else HID
    odt = _i32 if packed else _f32
    out = pl.pallas_call(
        functools.partial(_proj_body, nout, packed),
        grid=grid,
        in_specs=[
            pl.BlockSpec((bn, kin), lambda i: (i, 0)),
            pl.BlockSpec((kin, nout * HID), lambda i: (0, 0)),
            pl.BlockSpec((1, nout * HID), lambda i: (0, 0)),
        ],
        out_specs=[pl.BlockSpec((bn, ow), lambda i: (i, 0))
                   for _ in range(nout)],
        out_shape=[jax.ShapeDtypeStruct((n, ow), odt) for _ in range(nout)],
    )(x, w, b)
    return out


def _unpack(ref):
    w = ref[...]
    a = jax.lax.bitcast_convert_type(jax.lax.shift_left(w, 16), _f32)
    b = jax.lax.bitcast_convert_type(w & jnp.int32(-65536), _f32)
    return a, b


def _edge_body(qg_ref, kg_ref, vg_ref, ea_ref, we_ref, sel_ref, spread_ref,
               pvx_ref):
    hw = HD // 2
    eproj = jnp.dot(ea_ref[...], we_ref[...], preferred_element_type=_f32)
    e0, e1 = eproj[:, :hw], eproj[:, hw:]
    q0, q1 = _unpack(qg_ref)
    k0, k1 = _unpack(kg_ref)
    prod0 = q0 * (k0 + e0)
    prod1 = q1 * (k1 + e1)
    sel = sel_ref[...]
    logits = (jnp.dot(prod0, sel[:hw], preferred_element_type=_f32)
              + jnp.dot(prod1, sel[hw:], preferred_element_type=_f32))
    p = jnp.exp(logits * (1.0 / np.sqrt(D)))
    pexp = jnp.dot(p, spread_ref[...], preferred_element_type=_f32)
    v0, v1 = _unpack(vg_ref)
    pv0 = (v0 + e0) * pexp[:, :hw]
    pv1 = (v1 + e1) * pexp[:, hw:]
    be = p.shape[0]
    pvx_ref[...] = jnp.concatenate(
        [pv0, pv1, p, jnp.zeros((be, 128 - 16), _f32)], axis=-1)


def _edge(qg, kg, vg, ea, we, sel, spread, be):
    e = qg.shape[0]
    grid = (e // be,)
    return pl.pallas_call(
        _edge_body,
        grid=grid,
        in_specs=[
            pl.BlockSpec((be, HD // 2), lambda i: (i, 0)),
            pl.BlockSpec((be, HD // 2), lambda i: (i, 0)),
            pl.BlockSpec((be, HD // 2), lambda i: (i, 0)),
            pl.BlockSpec((be, ED), lambda i: (i, 0)),
            pl.BlockSpec((ED, HD), lambda i: (0, 0)),
            pl.BlockSpec((HD, 16), lambda i: (0, 0)),
            pl.BlockSpec((16, HD), lambda i: (0, 0)),
        ],
        out_specs=pl.BlockSpec((be, HD + 128), lambda i: (i, 0)),
        out_shape=jax.ShapeDtypeStruct((e, HD + 128), _f32),
    )(qg, kg, vg, ea, we, sel, spread)


def _combine_body(h_ref, ao_ref, so_ref, ac_ref, sc_ref, wo_ref,
                  spread_ref, lns_ref, lnb_ref, out_ref):
    spread = spread_ref[...]
    so = jnp.dot((so_ref[0] + so_ref[1])[:, :16], spread,
                 preferred_element_type=_f32)
    sc = jnp.dot((sc_ref[0] + sc_ref[1])[:, :16], spread,
                 preferred_element_type=_f32)
    m = (ao_ref[0] + ao_ref[1]) / (so + 1e-9) \
        + (ac_ref[0] + ac_ref[1]) / (sc + 1e-9)
    z = h_ref[...] + jnp.dot(m, wo_ref[...], preferred_element_type=_f32)
    mu = jnp.mean(z, axis=-1, keepdims=True)
    zc = z - mu
    var = jnp.mean(zc * zc, axis=-1, keepdims=True)
    out_ref[...] = zc * lax.rsqrt(var + 1e-5) * lns_ref[...] + lnb_ref[...]


def _combine(h, agg_o, s_o, agg_c, s_c, wo, spread, lns, lnb, bn):
    n = h.shape[0]
    grid = (n // bn,)
    return pl.pallas_call(
        _combine_body,
        grid=grid,
        in_specs=[
            pl.BlockSpec((bn, HID), lambda i: (i, 0)),
            pl.BlockSpec((2, bn, HD), lambda i: (0, i, 0)),
            pl.BlockSpec((2, bn, 128), lambda i: (0, i, 0)),
            pl.BlockSpec((2, bn, HD), lambda i: (0, i, 0)),
            pl.BlockSpec((2, bn, 128), lambda i: (0, i, 0)),
            pl.BlockSpec((HD, HID), lambda i: (0, 0)),
            pl.BlockSpec((16, HD), lambda i: (0, 0)),
            pl.BlockSpec((1, HID), lambda i: (0, 0)),
            pl.BlockSpec((1, HID), lambda i: (0, 0)),
        ],
        out_specs=pl.BlockSpec((bn, HID), lambda i: (i, 0)),
        out_shape=jax.ShapeDtypeStruct((n, HID), _f32),
    )(h, agg_o, s_o, agg_c, s_c, wo, spread, lns, lnb)


@functools.lru_cache(maxsize=None)
def _make_gather(e):
    per_w = e // NW
    c = 64
    n2 = per_w // (2 * c)
    hw = HD // 2
    mesh = plsc.VectorSubcoreMesh(core_axis_name="c", subcore_axis_name="s")

    @functools.partial(
        pl.kernel,
        out_type=[jax.ShapeDtypeStruct((e, hw), _i32)] * 3,
        mesh=mesh,
        scratch_types=[
            pltpu.VMEM((c,), _i32), pltpu.VMEM((c,), _i32),
            pltpu.VMEM((c,), _i32), pltpu.VMEM((c,), _i32),
            pltpu.VMEM((c, hw), _i32), pltpu.VMEM((c, hw), _i32),
            pltpu.VMEM((c, hw), _i32),
            pltpu.VMEM((c, hw), _i32), pltpu.VMEM((c, hw), _i32),
            pltpu.VMEM((c, hw), _i32),
            pltpu.SemaphoreType.DMA, pltpu.SemaphoreType.DMA,
            pltpu.SemaphoreType.DMA, pltpu.SemaphoreType.DMA,
            pltpu.SemaphoreType.DMA, pltpu.SemaphoreType.DMA,
        ],
    )
    def gather(qtab, ktab, vtab, dst, src, qg, kg, vg,
               d0, s0, d1, s1, q0, k0, v0, q1, k1, v1,
               sq0, sk0, sv0, sq1, sk1, sv1):
        wid = lax.axis_index("c") * 16 + lax.axis_index("s")
        base = wid * per_w
        dbuf = (d0, d1)
        sbuf = (s0, s1)
        rows = ((q0, k0, v0), (q1, k1, v1))
        sems = ((sq0, sk0, sv0), (sq1, sk1, sv1))

        def load_idx(off, b):
            pltpu.sync_copy(dst.at[pl.ds(off, c)], dbuf[b])
            pltpu.sync_copy(src.at[pl.ds(off, c)], sbuf[b])

        def issue(b):
            pltpu.async_copy(qtab.at[dbuf[b]], rows[b][0], sems[b][0])
            pltpu.async_copy(ktab.at[sbuf[b]], rows[b][1], sems[b][1])
            pltpu.async_copy(vtab.at[sbuf[b]], rows[b][2], sems[b][2])

        def wait_and_write(off, b):
            pltpu.make_async_copy(qtab.at[dbuf[b]], rows[b][0], sems[b][0]).wait()
            pltpu.make_async_copy(ktab.at[sbuf[b]], rows[b][1], sems[b][1]).wait()
            pltpu.make_async_copy(vtab.at[sbuf[b]], rows[b][2], sems[b][2]).wait()
            pltpu.sync_copy(rows[b][0], qg.at[pl.ds(off, c)])
            pltpu.sync_copy(rows[b][1], kg.at[pl.ds(off, c)])
            pltpu.sync_copy(rows[b][2], vg.at[pl.ds(off, c)])

        load_idx(base, 0)
        issue(0)

        def body(j, carry):
            off0 = base + (2 * j) * c
            load_idx(off0 + c, 1)
            issue(1)
            wait_and_write(off0, 0)

            @pl.when(j + 1 < n2)
            def _():
                load_idx(off0 + 2 * c, 0)
                issue(0)

            wait_and_write(off0 + c, 1)
            return carry

        lax.fori_loop(0, n2, body, 0)

    return gather


@functools.lru_cache(maxsize=None)
def _make_scatter(e):
    per_w = e // NW
    c = 64
    n_chunks = per_w // c
    rows_per_tile = K // 16
    hw = 128
    mesh = plsc.VectorSubcoreMesh(core_axis_name="c", subcore_axis_name="s")

    @functools.partial(
        pl.kernel,
        out_type=[
            jax.ShapeDtypeStruct((2, K, HD), _f32),
            jax.ShapeDtypeStruct((2, K, hw), _f32),
        ],
        mesh=mesh,
        scratch_types=[
            pltpu.VMEM((c,), _i32), pltpu.VMEM((c,), _i32),
            pltpu.VMEM((c, hw), _f32), pltpu.VMEM((c, hw), _f32),
            pltpu.VMEM((c, hw), _f32), pltpu.VMEM((c, hw), _f32),
            pltpu.VMEM_SHARED((K, hw), _f32),
            pltpu.VMEM_SHARED((K, hw), _f32),
            pltpu.SemaphoreType.DMA, pltpu.SemaphoreType.DMA,
            pltpu.SemaphoreType.DMA, pltpu.SemaphoreType.DMA,
            pltpu.SemaphoreType.DMA, pltpu.SemaphoreType.DMA,
        ],
    )
    def scatter(pvx, dst, zero_big, agg2, s2,
                d0, d1, a0, b0, a1, b1, acc0, acc1,
                sd0, sa0, sb0, sd1, sa1, sb1):
        cid = lax.axis_index("c")
        sid = lax.axis_index("s")
        wid = cid * 16 + sid
        row0 = sid * rows_per_tile
        base = wid * per_w
        dbuf = (d0, d1)
        bufa = (a0, a1)
        bufb = (b0, b1)
        sems = ((sd0, sa0, sb0), (sd1, sa1, sb1))
        n2 = n_chunks // 2

        def run_phase(groups, accs):
            ng = len(groups)

            def load(off, s):
                pltpu.async_copy(dst.at[pl.ds(off, c)], dbuf[s], sems[s][0])
                pltpu.async_copy(pvx.at[pl.ds(off, c),
                                        pl.ds(groups[0] * hw, hw)],
                                 bufa[s], sems[s][1])
                if ng > 1:
                    pltpu.async_copy(pvx.at[pl.ds(off, c),
                                            pl.ds(groups[1] * hw, hw)],
                                     bufb[s], sems[s][2])

            def wait_and_add(off, s):
                pltpu.make_async_copy(dst.at[pl.ds(off, c)], dbuf[s],
                                      sems[s][0]).wait()
                pltpu.make_async_copy(pvx.at[pl.ds(off, c),
                                             pl.ds(groups[0] * hw, hw)],
                                      bufa[s], sems[s][1]).wait()
                if ng > 1:
                    pltpu.make_async_copy(pvx.at[pl.ds(off, c),
                                                 pl.ds(groups[1] * hw, hw)],
                                          bufb[s], sems[s][2]).wait()
                pltpu.sync_copy(bufa[s], accs[0].at[dbuf[s]], add=True)
                if ng > 1:
                    pltpu.sync_copy(bufb[s], accs[1].at[dbuf[s]], add=True)

            load(base, 0)

            def body(j, carry):
                off0 = base + (2 * j) * c
                load(off0 + c, 1)
                wait_and_add(off0, 0)

                @pl.when(j + 1 < n2)
                def _():
                    load(off0 + 2 * c, 0)

                wait_and_add(off0 + c, 1)
                return carry

            lax.fori_loop(0, n2, body, 0)

        pltpu.sync_copy(zero_big, acc0.at[pl.ds(row0, rows_per_tile)])
        pltpu.sync_copy(zero_big, acc1.at[pl.ds(row0, rows_per_tile)])
        plsc.subcore_barrier()
        run_phase((0, 1), (acc0, acc1))
        plsc.subcore_barrier()
        pltpu.sync_copy(acc0.at[pl.ds(row0, rows_per_tile)],
                        agg2.at[cid, pl.ds(row0, rows_per_tile), pl.ds(0, hw)])
        pltpu.sync_copy(acc1.at[pl.ds(row0, rows_per_tile)],
                        agg2.at[cid, pl.ds(row0, rows_per_tile), pl.ds(hw, hw)])
        plsc.subcore_barrier()

        pltpu.sync_copy(zero_big, acc0.at[pl.ds(row0, rows_per_tile)])
        plsc.subcore_barrier()
        run_phase((2,), (acc0,))
        plsc.subcore_barrier()
        pltpu.sync_copy(acc0.at[pl.ds(row0, rows_per_tile)],
                        s2.at[cid, pl.ds(row0, rows_per_tile)])

    return scatter


def _conv(q, ktab, vtab, src, dst, ea, we, sel, spread, zb):
    e = src.shape[0]
    qg, kg, vg = _make_gather(e)(q, ktab, vtab, dst, src)
    pvx = _edge(qg, kg, vg, ea, we, sel, spread, be=1024)
    agg2, s2 = _make_scatter(e)(pvx, dst, zb)
    return agg2, s2


def kernel(scene_x, gripper_left_x, gripper_right_x, edge_index_obs_left,
           edge_index_obs_right, edge_index_coord_lr, edge_index_coord_rl,
           edge_attr_obs_left, edge_attr_obs_right, edge_attr_coord_lr,
           edge_attr_coord_rl, scene_W, scene_b, grip_W, grip_b,
           Wq, Wk, Wv, We, Wo, ln_s, ln_b):
    sel = jnp.zeros((HD, 16), _f32).at[jnp.arange(HD), jnp.arange(HD) // D].set(1.0)
    spread = sel.T[:, :]
    zb = jnp.zeros((K // 16, 128), _f32)

    src_ol, dst_ol = edge_index_obs_left[0], edge_index_obs_left[1]
    src_or, dst_or = edge_index_obs_right[0], edge_index_obs_right[1]
    src_lr, dst_lr = edge_index_coord_lr[0], edge_index_coord_lr[1]
    src_rl, dst_rl = edge_index_coord_rl[0], edge_index_coord_rl[1]

    (scene,) = _proj(scene_x, scene_W, scene_b[None, :], 1, bn=400)
    (left,) = _proj(gripper_left_x, grip_W, grip_b[None, :], 1, bn=512)
    (right,) = _proj(gripper_right_x, grip_W, grip_b[None, :], 1, bn=512)

    zbias1 = jnp.zeros((1, 4 * HID), _f32)
    for l in range(L):
        k0, v0, k1, v1 = _proj(
            scene, jnp.concatenate([Wk[l, 0], Wv[l, 0], Wk[l, 1], Wv[l, 1]], axis=1),
            zbias1, 4, bn=400, packed=True)
        q0, q3, k2, v2 = _proj(
            left, jnp.concatenate([Wq[l, 0], Wq[l, 3], Wk[l, 2], Wv[l, 2]], axis=1),
            zbias1, 4, bn=512, packed=True)
        q1, q2, k3, v3 = _proj(
            right, jnp.concatenate([Wq[l, 1], Wq[l, 2], Wk[l, 3], Wv[l, 3]], axis=1),
            zbias1, 4, bn=512, packed=True)

        ao_l, so_l = _conv(q0, k0, v0, src_ol, dst_ol, edge_attr_obs_left,
                           We[l, 0], sel, spread, zb)
        ac_l, sc_l = _conv(q3, k3, v3, src_rl, dst_rl, edge_attr_coord_rl,
                           We[l, 3], sel, spread, zb)
        ao_r, so_r = _conv(q1, k1, v1, src_or, dst_or, edge_attr_obs_right,
                           We[l, 1], sel, spread, zb)
        ac_r, sc_r = _conv(q2, k2, v2, src_lr, dst_lr, edge_attr_coord_lr,
                           We[l, 2], sel, spread, zb)

        new_left = _combine(left, ao_l, so_l, ac_l, sc_l, Wo[l, 1], spread,
                            ln_s[l, 1][None, :], ln_b[l, 1][None, :], bn=512)
        new_right = _combine(right, ao_r, so_r, ac_r, sc_r, Wo[l, 2], spread,
                             ln_s[l, 2][None, :], ln_b[l, 2][None, :], bn=512)
        left, right = new_left, new_right

    return (left, right)
